# baseline (speedup 1.0000x reference)
import functools

import jax
import jax.numpy as jnp
from jax import lax
from jax.experimental import pallas as pl
from jax.experimental.pallas import tpu as pltpu
from jax.experimental.pallas import tpu_sc as plsc

BATCH = 1024
CAST = 256
HID = 128
INP = 128
NROWS = BATCH * CAST
NW = 32
B_PER_W = BATCH // NW
ROWS_PER_W = NROWS // NW
CH = 256
NCH = ROWS_PER_W // CH


def _sc_gather():
    mesh = plsc.VectorSubcoreMesh(core_axis_name="c", subcore_axis_name="s")

    @functools.partial(
        pl.kernel,
        out_type=jax.ShapeDtypeStruct((BATCH, HID), jnp.float32),
        mesh=mesh,
        scratch_types=[
            pltpu.VMEM((B_PER_W,), jnp.int32),
            pltpu.VMEM((B_PER_W, HID), jnp.float32),
            pltpu.SemaphoreType.DMA,
        ],
    )
    def body(st_hbm, idx_hbm, out_hbm, idx_v, rows_v, sem):
        wid = lax.axis_index("s") * 2 + lax.axis_index("c")
        base = wid * B_PER_W
        pltpu.sync_copy(idx_hbm.at[pl.ds(base, B_PER_W)], idx_v)
        pltpu.async_copy(st_hbm.at[idx_v], rows_v, sem).wait()
        pltpu.sync_copy(rows_v, out_hbm.at[pl.ds(base, B_PER_W)])

    return body


def _gru_body(x_ref, h_ref, wi_ref, wh_ref, bi_ref, bh_ref, stop_ref,
              sel_ref, scat_ref, flag_ref):
    x = x_ref[...]
    h = h_ref[...]
    dn = (((1,), (1,)), ((), ()))
    gi = lax.dot_general(x, wi_ref[...], dn,
                         preferred_element_type=jnp.float32) + bi_ref[...]
    gh = lax.dot_general(h, wh_ref[...], dn,
                         preferred_element_type=jnp.float32) + bh_ref[...]
    i_r, i_z, i_n = gi[:, :HID], gi[:, HID:2 * HID], gi[:, 2 * HID:]
    h_r, h_z, h_n = gh[:, :HID], gh[:, HID:2 * HID], gh[:, 2 * HID:]
    r = jax.nn.sigmoid(i_r + h_r)
    z = jax.nn.sigmoid(i_z + h_z)
    n = jnp.tanh(i_n + r * h_n)
    new_h = (1.0 - z) * n + z * h
    sel_ref[...] = new_h
    rid = lax.broadcasted_iota(jnp.int32, (BATCH, 1), 0)
    stopped = jnp.any(rid == stop_ref[...], axis=1, keepdims=True)
    scat_ref[...] = jnp.where(stopped, 0.0, new_h)
    flag_ref[...] = stopped.astype(jnp.int32)


def _sc_update():
    mesh = plsc.VectorSubcoreMesh(core_axis_name="c", subcore_axis_name="s")

    @functools.partial(
        pl.kernel,
        out_type=jax.ShapeDtypeStruct((NROWS, HID), jnp.float32),
        mesh=mesh,
        scratch_types=[
            pltpu.VMEM((CH, HID), jnp.float32),
            pltpu.VMEM((CH, HID), jnp.float32),
            pltpu.VMEM((B_PER_W,), jnp.int32),
            pltpu.VMEM((B_PER_W, HID), jnp.float32),
            pltpu.VMEM((B_PER_W,), jnp.int32),
            pltpu.SemaphoreType.DMA,
            pltpu.SemaphoreType.DMA,
            pltpu.SemaphoreType.DMA,
            pltpu.SemaphoreType.DMA,
            pltpu.SemaphoreType.DMA,
        ],
    )
    def body(st_hbm, rows_hbm, idx_hbm, flag_hbm, zeros_hbm, out_hbm,
             buf0, buf1, idx_v, rows_v, flags_v, si0, si1, so0, so1, sx):
        wid = lax.axis_index("s") * 2 + lax.axis_index("c")
        base = wid * ROWS_PER_W
        bufs = (buf0, buf1)
        sin = (si0, si1)
        sout = (so0, so1)
        rbase = wid * B_PER_W
        pltpu.sync_copy(idx_hbm.at[pl.ds(rbase, B_PER_W)], idx_v)
        pltpu.sync_copy(rows_hbm.at[pl.ds(rbase, B_PER_W)], rows_v)
        pltpu.sync_copy(flag_hbm.at[pl.ds(rbase, B_PER_W)], flags_v)
        for k in range(NCH):
            b = k % 2
            if k >= 2:
                pltpu.make_async_copy(
                    bufs[b], out_hbm.at[pl.ds(base + (k - 2) * CH, CH)],
                    sout[b]).wait()
            cin = pltpu.make_async_copy(
                st_hbm.at[pl.ds(base + k * CH, CH)], bufs[b], sin[b])
            cin.start()
            cin.wait()
            pltpu.make_async_copy(
                bufs[b], out_hbm.at[pl.ds(base + k * CH, CH)], sout[b]).start()
        for k in (NCH - 2, NCH - 1):
            b = k % 2
            pltpu.make_async_copy(
                bufs[b], out_hbm.at[pl.ds(base + k * CH, CH)], sout[b]).wait()
        pltpu.async_copy(rows_v, out_hbm.at[idx_v], sx).wait()
        for g2 in range(B_PER_W // 16):
            fvec = flags_v[pl.ds(g2 * 16, 16)]
            for j in range(16):
                row = rbase + g2 * 16 + j
                cond = fvec[j] != 0

                def _zero(row=row):
                    pltpu.sync_copy(zeros_hbm,
                                    out_hbm.at[pl.ds(row * CAST, CAST)])
                pl.when(cond)(_zero)

    return body


def kernel(x, state, W_ih, W_hh, b_ih, b_hh, batch_idxs, actor_ids,
           story_stop_idxs):
    del batch_idxs
    aid = jnp.clip(actor_ids, 0, CAST - 1).astype(jnp.int32)
    flat_idx = jnp.arange(BATCH, dtype=jnp.int32) * CAST + aid
    stops = story_stop_idxs.astype(jnp.int32)
    zeros = jnp.zeros((CAST, HID), jnp.float32)
    st2 = state.reshape(NROWS, HID)

    h = _sc_gather()(st2, flat_idx)

    new_selected, scat_rows, stop_flags = pl.pallas_call(
        _gru_body,
        in_specs=[
            pl.BlockSpec((BATCH, INP), lambda: (0, 0)),
            pl.BlockSpec((BATCH, HID), lambda: (0, 0)),
            pl.BlockSpec((3 * HID, INP), lambda: (0, 0)),
            pl.BlockSpec((3 * HID, HID), lambda: (0, 0)),
            pl.BlockSpec((1, 3 * HID), lambda: (0, 0)),
            pl.BlockSpec((1, 3 * HID), lambda: (0, 0)),
            pl.BlockSpec((1, 16), lambda: (0, 0)),
        ],
        out_specs=[
            pl.BlockSpec((BATCH, HID), lambda: (0, 0)),
            pl.BlockSpec((BATCH, HID), lambda: (0, 0)),
            pl.BlockSpec((BATCH, 1), lambda: (0, 0)),
        ],
        out_shape=[
            jax.ShapeDtypeStruct((BATCH, HID), jnp.float32),
            jax.ShapeDtypeStruct((BATCH, HID), jnp.float32),
            jax.ShapeDtypeStruct((BATCH, 1), jnp.int32),
        ],
    )(x, h, W_ih, W_hh, b_ih.reshape(1, -1), b_hh.reshape(1, -1),
      stops.reshape(1, 16))

    new_state = _sc_update()(st2, scat_rows, flat_idx,
                             stop_flags.reshape(BATCH), zeros)
    return new_selected, new_state.reshape(BATCH, CAST, HID)


# speedup vs baseline: 2.0992x; 2.0992x over previous
import jax
import jax.numpy as jnp
from jax.experimental import pallas as pl
from jax.experimental.pallas import tpu as pltpu

BATCH = 1024
CAST = 256
HID = 128
INP = 128
BLK = 64


def _fused_body(x_ref, st_ref, wiT_ref, whT_ref, bi_ref, bh_ref, aid_ref,
                stop_ref, sel_ref, out_ref):
    g = pl.program_id(0)
    base = g * BLK
    x = x_ref[...]

    rows = []
    for r_i in range(BLK):
        a = jnp.clip(aid_ref[base + r_i], 0, CAST - 1)
        rows.append(st_ref[r_i, pl.ds(a, 1), :])
    h = jnp.concatenate(rows, axis=0)

    dn = (((1,), (1,)), ((), ()))
    gi = jax.lax.dot_general(x, wiT_ref[...], dn,
                             preferred_element_type=jnp.float32) + bi_ref[...]
    gh = jax.lax.dot_general(h, whT_ref[...], dn,
                             preferred_element_type=jnp.float32) + bh_ref[...]
    i_r, i_z, i_n = gi[:, :HID], gi[:, HID:2 * HID], gi[:, 2 * HID:]
    h_r, h_z, h_n = gh[:, :HID], gh[:, HID:2 * HID], gh[:, 2 * HID:]
    r = jax.nn.sigmoid(i_r + h_r)
    z = jax.nn.sigmoid(i_z + h_z)
    n = jnp.tanh(i_n + r * h_n)
    new_h = (1.0 - z) * n + z * h
    sel_ref[...] = new_h

    out_ref[...] = st_ref[...]
    for r_i in range(BLK):
        a = jnp.clip(aid_ref[base + r_i], 0, CAST - 1)
        out_ref[r_i, pl.ds(a, 1), :] = new_h[r_i:r_i + 1, :]
    n_stop = stop_ref.shape[0]
    for r_i in range(BLK):
        rid = base + r_i
        cond = stop_ref[0] == rid
        for j in range(1, n_stop):
            cond = jnp.logical_or(cond, stop_ref[j] == rid)

        def _zero(r_i=r_i):
            out_ref[r_i] = jnp.zeros((CAST, HID), jnp.float32)
        pl.when(cond)(_zero)


def kernel(x, state, W_ih, W_hh, b_ih, b_hh, batch_idxs, actor_ids,
           story_stop_idxs):
    del batch_idxs
    aid = actor_ids.astype(jnp.int32)
    stops = story_stop_idxs.astype(jnp.int32)
    bi = b_ih.reshape(1, 3 * HID)
    bh = b_hh.reshape(1, 3 * HID)

    grid = BATCH // BLK
    new_selected, new_state = pl.pallas_call(
        _fused_body,
        grid=(grid,),
        in_specs=[
            pl.BlockSpec((BLK, INP), lambda g: (g, 0)),
            pl.BlockSpec((BLK, CAST, HID), lambda g: (g, 0, 0)),
            pl.BlockSpec((3 * HID, INP), lambda g: (0, 0)),
            pl.BlockSpec((3 * HID, HID), lambda g: (0, 0)),
            pl.BlockSpec((1, 3 * HID), lambda g: (0, 0)),
            pl.BlockSpec((1, 3 * HID), lambda g: (0, 0)),
            pl.BlockSpec(memory_space=pltpu.SMEM),
            pl.BlockSpec(memory_space=pltpu.SMEM),
        ],
        out_specs=[
            pl.BlockSpec((BLK, HID), lambda g: (g, 0)),
            pl.BlockSpec((BLK, CAST, HID), lambda g: (g, 0, 0)),
        ],
        out_shape=[
            jax.ShapeDtypeStruct((BATCH, HID), jnp.float32),
            jax.ShapeDtypeStruct((BATCH, CAST, HID), jnp.float32),
        ],
    )(x, state, W_ih, W_hh, bi, bh, aid, stops)
    return new_selected, new_state
